# Initial kernel scaffold; baseline (speedup 1.0000x reference)
import jax
import jax.numpy as jnp
from jax.experimental import pallas as pl


def kernel(inputs):
    raise NotImplementedError("write your pallas kernel here")


# speedup vs baseline: 2.1068x; 2.1068x over previous
import functools

import jax
import jax.numpy as jnp
from jax import lax
from jax.experimental import pallas as pl
from jax.experimental.pallas import tpu as pltpu
from jax.experimental.pallas import tpu_sc as plsc

N_ROWS = 128
ROW = 32768
L = 16
NVEC = ROW // L
NUM_CORES = 2
NUM_SUBCORES = 16
WORKERS = NUM_CORES * NUM_SUBCORES
ROWS_PER_W = N_ROWS // WORKERS


def _clear(hist, nvec):
    zeros = jnp.zeros((L,), jnp.int32)

    def body(i, c):
        hist[pl.ds(i * L, L)] = zeros
        return c

    lax.fori_loop(0, nvec, body, 0)


def _excl_prefix(hist, nvec):
    def body(i, carry):
        h = hist[pl.ds(i * L, L)]
        inc = plsc.cumsum(h)
        hist[pl.ds(i * L, L)] = inc - h + carry
        return carry + jnp.max(inc)

    lax.fori_loop(0, nvec, body, jnp.int32(0))


def _sc_body(in_hbm, out_hbm, buf_a, buf_b, hist):
    cid = lax.axis_index("c")
    sid = lax.axis_index("s")
    wid = sid * NUM_CORES + cid
    lane = lax.iota(jnp.int32, L)
    ones = jnp.ones((L,), jnp.int32)

    def do_row(j, c0):
        r = wid * ROWS_PER_W + j
        pltpu.sync_copy(in_hbm.at[r], buf_a)

        _clear(hist, NVEC)

        def h1(i, c):
            v = buf_a[pl.ds(i * L, L)]
            u = plsc.bitcast(v, jnp.int32)
            m = lax.shift_right_arithmetic(u, 31)
            k = u ^ ((m ^ jnp.int32(-1)) & jnp.int32(0x7FFFFFFF))
            buf_a[pl.ds(i * L, L)] = plsc.bitcast(k, jnp.float32)
            d = k & jnp.int32(0x7FFF)
            plsc.addupdate_scatter(hist, [d], ones)
            return c

        lax.fori_loop(0, NVEC, h1, 0)
        _excl_prefix(hist, NVEC)

        def s1(i, c):
            k = plsc.bitcast(buf_a[pl.ds(i * L, L)], jnp.int32)
            d = k & jnp.int32(0x7FFF)
            cnt, _ = plsc.scan_count(d)
            pos = plsc.load_gather(hist, [d]) + cnt - 1
            p = (k & jnp.int32(-32768)) | (i * L + lane)
            plsc.store_scatter(buf_b, [pos], p)
            plsc.addupdate_scatter(hist, [d], ones)
            return c

        lax.fori_loop(0, NVEC, s1, 0)

        _clear(hist, 512 // L)

        def h2(i, c):
            p = buf_b[pl.ds(i * L, L)]
            d = lax.shift_right_logical(p, 15) & jnp.int32(0x1FF)
            plsc.addupdate_scatter(hist, [d], ones)
            return c

        lax.fori_loop(0, NVEC, h2, 0)
        _excl_prefix(hist, 512 // L)

        def s2(i, c):
            p = buf_b[pl.ds(i * L, L)]
            d = lax.shift_right_logical(p, 15) & jnp.int32(0x1FF)
            cnt, _ = plsc.scan_count(d)
            pos = plsc.load_gather(hist, [d]) + cnt - 1
            plsc.store_scatter(buf_a, [pos], plsc.bitcast(p, jnp.float32))
            plsc.addupdate_scatter(hist, [d], ones)
            return c

        lax.fori_loop(0, NVEC, s2, 0)

        _clear(hist, 256 // L)

        def h3(i, c):
            p = plsc.bitcast(buf_a[pl.ds(i * L, L)], jnp.int32)
            d = lax.shift_right_logical(p, 24) & jnp.int32(0xFF)
            plsc.addupdate_scatter(hist, [d], ones)
            return c

        lax.fori_loop(0, NVEC, h3, 0)
        _excl_prefix(hist, 256 // L)

        def s3(i, c):
            p = plsc.bitcast(buf_a[pl.ds(i * L, L)], jnp.int32)
            d = lax.shift_right_logical(p, 24) & jnp.int32(0xFF)
            cnt, _ = plsc.scan_count(d)
            pos = plsc.load_gather(hist, [d]) + cnt - 1
            plsc.store_scatter(buf_b, [pos], p & jnp.int32(0x7FFF))
            plsc.addupdate_scatter(hist, [d], ones)
            return c

        lax.fori_loop(0, NVEC, s3, 0)

        pltpu.sync_copy(buf_b, out_hbm.at[r])
        return c0

    lax.fori_loop(0, ROWS_PER_W, do_row, 0)


_argsort_desc = functools.partial(
    pl.kernel,
    out_type=jax.ShapeDtypeStruct((N_ROWS, ROW), jnp.int32),
    mesh=plsc.VectorSubcoreMesh(core_axis_name="c", subcore_axis_name="s"),
    scratch_types=[
        pltpu.VMEM((ROW,), jnp.float32),
        pltpu.VMEM((ROW,), jnp.int32),
        pltpu.VMEM((ROW,), jnp.int32),
    ],
    compiler_params=pltpu.CompilerParams(needs_layout_passes=False),
)(_sc_body)


@jax.jit
def kernel(inputs):
    return _argsort_desc(inputs)


# speedup vs baseline: 4.1510x; 1.9703x over previous
import functools

import jax
import jax.numpy as jnp
from jax import lax
from jax.experimental import pallas as pl
from jax.experimental.pallas import tpu as pltpu
from jax.experimental.pallas import tpu_sc as plsc

N_ROWS = 128
ROW = 32768
L = 16
NVEC = ROW // L
NUM_CORES = 2
NUM_SUBCORES = 16
WORKERS = NUM_CORES * NUM_SUBCORES
ROWS_PER_W = N_ROWS // WORKERS

_ONES = None


def _clear(hist, nvec, unroll=8):
    zeros = jnp.zeros((L,), jnp.int32)

    def body(i, c):
        hist[pl.ds(i * L, L)] = zeros
        return c

    lax.fori_loop(0, nvec, body, 0, unroll=unroll)


def _excl_prefix(hist, nvec, unroll=4):
    def body(i, carry):
        h = hist[pl.ds(i * L, L)]
        inc = plsc.cumsum(h)
        hist[pl.ds(i * L, L)] = inc - h + carry
        return carry + jnp.sum(h)

    lax.fori_loop(0, nvec, body, jnp.int32(0), unroll=unroll)


def _scatter_pass(src_load, digit_fn, payload_fn, store_fn, hist, ones,
                  unroll=2):
    def stage(i):
        x = src_load(i)
        d = digit_fn(x)
        cnt, _ = plsc.scan_count(d)
        return d, cnt, payload_fn(x, i)

    def commit(d, cnt, p):
        pos = plsc.load_gather(hist, [d]) + cnt - 1
        store_fn(pos, p)
        plsc.addupdate_scatter(hist, [d], ones)

    def body(i, carry):
        nxt = stage(i + 1)
        commit(*carry)
        return nxt

    last = lax.fori_loop(0, NVEC - 1, body, stage(0), unroll=unroll)
    commit(*last)


def _sc_body(in_hbm, out_hbm, buf_a, buf_b, hist1, hist2, hist3):
    cid = lax.axis_index("c")
    sid = lax.axis_index("s")
    wid = sid * NUM_CORES + cid
    lane = lax.iota(jnp.int32, L)
    ones = jnp.ones((L,), jnp.int32)

    def do_row(j, c0):
        r = wid * ROWS_PER_W + j
        pltpu.sync_copy(in_hbm.at[r], buf_a)

        _clear(hist1, NVEC)
        _clear(hist2, 512 // L)
        _clear(hist3, 256 // L)

        def hall(i, c):
            v = buf_a[pl.ds(i * L, L)]
            u = plsc.bitcast(v, jnp.int32)
            m = lax.shift_right_arithmetic(u, 31)
            k = u ^ ((m ^ jnp.int32(-1)) & jnp.int32(0x7FFFFFFF))
            buf_a[pl.ds(i * L, L)] = plsc.bitcast(k, jnp.float32)
            plsc.addupdate_scatter(hist1, [k & jnp.int32(0x7FFF)], ones)
            plsc.addupdate_scatter(
                hist2, [lax.shift_right_logical(k, 15) & jnp.int32(0x1FF)],
                ones)
            plsc.addupdate_scatter(
                hist3, [lax.shift_right_logical(k, 24) & jnp.int32(0xFF)],
                ones)
            return c

        lax.fori_loop(0, NVEC, hall, 0, unroll=2)

        _excl_prefix(hist1, NVEC)
        _excl_prefix(hist2, 512 // L)
        _excl_prefix(hist3, 256 // L)

        _scatter_pass(
            src_load=lambda i: plsc.bitcast(buf_a[pl.ds(i * L, L)], jnp.int32),
            digit_fn=lambda k: k & jnp.int32(0x7FFF),
            payload_fn=lambda k, i: (k & jnp.int32(-32768)) | (i * L + lane),
            store_fn=lambda pos, p: plsc.store_scatter(buf_b, [pos], p),
            hist=hist1, ones=ones)

        _scatter_pass(
            src_load=lambda i: buf_b[pl.ds(i * L, L)],
            digit_fn=lambda p: lax.shift_right_logical(p, 15)
            & jnp.int32(0x1FF),
            payload_fn=lambda p, i: p,
            store_fn=lambda pos, p: plsc.store_scatter(
                buf_a, [pos], plsc.bitcast(p, jnp.float32)),
            hist=hist2, ones=ones)

        _scatter_pass(
            src_load=lambda i: plsc.bitcast(buf_a[pl.ds(i * L, L)], jnp.int32),
            digit_fn=lambda p: lax.shift_right_logical(p, 24)
            & jnp.int32(0xFF),
            payload_fn=lambda p, i: p & jnp.int32(0x7FFF),
            store_fn=lambda pos, p: plsc.store_scatter(buf_b, [pos], p),
            hist=hist3, ones=ones)

        pltpu.sync_copy(buf_b, out_hbm.at[r])
        return c0

    lax.fori_loop(0, ROWS_PER_W, do_row, 0)


_argsort_desc = functools.partial(
    pl.kernel,
    out_type=jax.ShapeDtypeStruct((N_ROWS, ROW), jnp.int32),
    mesh=plsc.VectorSubcoreMesh(core_axis_name="c", subcore_axis_name="s"),
    scratch_types=[
        pltpu.VMEM((ROW,), jnp.float32),
        pltpu.VMEM((ROW,), jnp.int32),
        pltpu.VMEM((ROW,), jnp.int32),
        pltpu.VMEM((512,), jnp.int32),
        pltpu.VMEM((256,), jnp.int32),
    ],
    compiler_params=pltpu.CompilerParams(needs_layout_passes=False),
)(_sc_body)


@jax.jit
def kernel(inputs):
    return _argsort_desc(inputs)


# speedup vs baseline: 4.2916x; 1.0339x over previous
import functools

import jax
import jax.numpy as jnp
from jax import lax
from jax.experimental import pallas as pl
from jax.experimental.pallas import tpu as pltpu
from jax.experimental.pallas import tpu_sc as plsc

N_ROWS = 128
ROW = 32768
L = 16
NVEC = ROW // L
NUM_CORES = 2
NUM_SUBCORES = 16
WORKERS = NUM_CORES * NUM_SUBCORES
ROWS_PER_W = N_ROWS // WORKERS

_ONES = None


def _clear(hist, nvec, unroll=16):
    zeros = jnp.zeros((L,), jnp.int32)

    def body(i, c):
        hist[pl.ds(i * L, L)] = zeros
        return c

    lax.fori_loop(0, nvec, body, 0, unroll=unroll)


def _excl_prefix(hist, nvec, unroll=8):
    def body(i, carry):
        h = hist[pl.ds(i * L, L)]
        inc = plsc.cumsum(h)
        hist[pl.ds(i * L, L)] = inc - h + carry
        return carry + jnp.sum(h)

    lax.fori_loop(0, nvec, body, jnp.int32(0), unroll=unroll)


def _scatter_pass(src_load, digit_fn, payload_fn, store_fn, hist, ones,
                  unroll=4):
    def stage(i):
        x = src_load(i)
        d = digit_fn(x)
        cnt, _ = plsc.scan_count(d)
        return d, cnt, payload_fn(x, i)

    def commit(d, cnt, p):
        pos = plsc.load_gather(hist, [d]) + cnt - 1
        store_fn(pos, p)
        plsc.addupdate_scatter(hist, [d], ones)

    def body(i, carry):
        nxt = stage(i + 1)
        commit(*carry)
        return nxt

    last = lax.fori_loop(0, NVEC - 1, body, stage(0), unroll=unroll)
    commit(*last)


def _sc_body(in_hbm, out_hbm, buf_a, buf_b, hist1, hist2, hist3):
    cid = lax.axis_index("c")
    sid = lax.axis_index("s")
    wid = sid * NUM_CORES + cid
    lane = lax.iota(jnp.int32, L)
    ones = jnp.ones((L,), jnp.int32)

    def do_row(j, c0):
        r = wid * ROWS_PER_W + j
        pltpu.sync_copy(in_hbm.at[r], buf_a)

        _clear(hist1, NVEC)
        _clear(hist2, 512 // L)
        _clear(hist3, 256 // L)

        def hall(i, c):
            v = buf_a[pl.ds(i * L, L)]
            u = plsc.bitcast(v, jnp.int32)
            m = lax.shift_right_arithmetic(u, 31)
            k = u ^ ((m ^ jnp.int32(-1)) & jnp.int32(0x7FFFFFFF))
            buf_a[pl.ds(i * L, L)] = plsc.bitcast(k, jnp.float32)
            plsc.addupdate_scatter(hist1, [k & jnp.int32(0x7FFF)], ones)
            plsc.addupdate_scatter(
                hist2, [lax.shift_right_logical(k, 15) & jnp.int32(0x1FF)],
                ones)
            plsc.addupdate_scatter(
                hist3, [lax.shift_right_logical(k, 24) & jnp.int32(0xFF)],
                ones)
            return c

        lax.fori_loop(0, NVEC, hall, 0, unroll=4)

        _excl_prefix(hist1, NVEC)
        _excl_prefix(hist2, 512 // L)
        _excl_prefix(hist3, 256 // L)

        _scatter_pass(
            src_load=lambda i: plsc.bitcast(buf_a[pl.ds(i * L, L)], jnp.int32),
            digit_fn=lambda k: k & jnp.int32(0x7FFF),
            payload_fn=lambda k, i: (k & jnp.int32(-32768)) | (i * L + lane),
            store_fn=lambda pos, p: plsc.store_scatter(buf_b, [pos], p),
            hist=hist1, ones=ones)

        _scatter_pass(
            src_load=lambda i: buf_b[pl.ds(i * L, L)],
            digit_fn=lambda p: lax.shift_right_logical(p, 15)
            & jnp.int32(0x1FF),
            payload_fn=lambda p, i: p,
            store_fn=lambda pos, p: plsc.store_scatter(
                buf_a, [pos], plsc.bitcast(p, jnp.float32)),
            hist=hist2, ones=ones)

        _scatter_pass(
            src_load=lambda i: plsc.bitcast(buf_a[pl.ds(i * L, L)], jnp.int32),
            digit_fn=lambda p: lax.shift_right_logical(p, 24)
            & jnp.int32(0xFF),
            payload_fn=lambda p, i: p & jnp.int32(0x7FFF),
            store_fn=lambda pos, p: plsc.store_scatter(buf_b, [pos], p),
            hist=hist3, ones=ones)

        pltpu.sync_copy(buf_b, out_hbm.at[r])
        return c0

    lax.fori_loop(0, ROWS_PER_W, do_row, 0)


_argsort_desc = functools.partial(
    pl.kernel,
    out_type=jax.ShapeDtypeStruct((N_ROWS, ROW), jnp.int32),
    mesh=plsc.VectorSubcoreMesh(core_axis_name="c", subcore_axis_name="s"),
    scratch_types=[
        pltpu.VMEM((ROW,), jnp.float32),
        pltpu.VMEM((ROW,), jnp.int32),
        pltpu.VMEM((ROW,), jnp.int32),
        pltpu.VMEM((512,), jnp.int32),
        pltpu.VMEM((256,), jnp.int32),
    ],
    compiler_params=pltpu.CompilerParams(needs_layout_passes=False),
)(_sc_body)


@jax.jit
def kernel(inputs):
    return _argsort_desc(inputs)
